# baseline (speedup 1.0000x reference)
import numpy as np
import jax
import jax.numpy as jnp
from jax.experimental import pallas as pl
from jax.experimental.pallas import tpu as pltpu

_FEAT_SIZES = [(64, 64), (32, 32), (16, 16), (8, 8), (4, 4)]
_NUM_CLASSES = 80
_A = 9
_IMG_H = 512.0
_IMG_W = 512.0
_NMS_PRE = 1000
_IOU_THR = 0.5
_SCORE_THR = 0.05
_MAX_PER_IMG = 100
_MAX_RATIO = float(np.abs(np.log(16.0 / 1000.0)))
_BIG_G = np.int32(2 ** 30)


def _level_buffers(l, cls, bb, an):
    h, w = _FEAT_SIZES[l]
    hw = h * w
    wide = hw >= 128
    if wide:
        rows = _A * (hw // 128)
        maxlog = jnp.max(cls, axis=1).reshape(rows, 128)
        dx = bb[:, 0].reshape(rows, 128)
        dy = bb[:, 1].reshape(rows, 128)
        dw = bb[:, 2].reshape(rows, 128)
        dh = bb[:, 3].reshape(rows, 128)
        ax1 = an[:, 0].reshape(rows, 128)
        ay1 = an[:, 1].reshape(rows, 128)
        ax2 = an[:, 2].reshape(rows, 128)
        ay2 = an[:, 3].reshape(rows, 128)
        shape = (rows, 128)
        ri = jax.lax.broadcasted_iota(jnp.int32, shape, 0)
        li = jax.lax.broadcasted_iota(jnp.int32, shape, 1)
        f = ri * 128 + li
        a_idx = f // hw
        rem = f - a_idx * hw
        p = rem * _A + a_idx
        valid = None
    else:
        rows = _A
        maxlog = jnp.max(cls, axis=1)
        dx, dy, dw, dh = bb[:, 0], bb[:, 1], bb[:, 2], bb[:, 3]
        ax1, ay1, ax2, ay2 = an[:, 0], an[:, 1], an[:, 2], an[:, 3]
        shape = (rows, hw)
        ri = jax.lax.broadcasted_iota(jnp.int32, shape, 0)
        li = jax.lax.broadcasted_iota(jnp.int32, shape, 1)
        p = li * _A + ri
        valid = None

    scores = jax.nn.sigmoid(maxlog)
    g = p + np.int32(l << 16)

    axc = (ax1 + ax2) * 0.5
    ayc = (ay1 + ay2) * 0.5
    aw = ax2 - ax1
    ah = ay2 - ay1
    dwc = jnp.clip(dw, -_MAX_RATIO, _MAX_RATIO)
    dhc = jnp.clip(dh, -_MAX_RATIO, _MAX_RATIO)
    cx = axc + aw * dx
    cy = ayc + ah * dy
    bw = aw * jnp.exp(dwc)
    bh = ah * jnp.exp(dhc)
    x1 = jnp.clip(cx - bw * 0.5, 0.0, _IMG_W)
    y1 = jnp.clip(cy - bh * 0.5, 0.0, _IMG_H)
    x2 = jnp.clip(cx + bw * 0.5, 0.0, _IMG_W)
    y2 = jnp.clip(cy + bh * 0.5, 0.0, _IMG_H)

    n = hw * _A
    k = min(_NMS_PRE, n)
    if k < n:
        keys = jax.lax.bitcast_convert_type(scores, jnp.int32)

        def bs_body(_, lohi):
            lo, hi = lohi
            mid = (lo + hi + 1) // 2
            c = jnp.sum((keys >= mid).astype(jnp.int32), keepdims=True)
            ge = c >= k
            return (jnp.where(ge, mid, lo), jnp.where(ge, hi, mid - 1))

        lo0 = jnp.zeros((1, 1), jnp.int32)
        hi0 = jnp.full((1, 1), 0x40000000, jnp.int32)
        T, _ = jax.lax.fori_loop(0, 31, bs_body, (lo0, hi0))

        strictly = keys > T
        m = jnp.sum(strictly.astype(jnp.int32), keepdims=True)
        need = k - m
        tie = keys == T

        def tie_body(_, lohi):
            lo, hi = lohi
            mid = (lo + hi) // 2
            c = jnp.sum((tie & (p <= mid)).astype(jnp.int32), keepdims=True)
            ge = c >= need
            return (jnp.where(ge, lo, mid + 1), jnp.where(ge, mid, hi))

        pstar, _ = jax.lax.fori_loop(
            0, 17, tie_body,
            (jnp.zeros((1, 1), jnp.int32), jnp.full((1, 1), n - 1, jnp.int32)))
        keep = strictly | (tie & (p <= pstar) & (need > 0))
        scores = jnp.where(keep, scores, -1.0)

    if not wide:
        pad = 128 - hw
        scores = jnp.concatenate(
            [scores, jnp.full((rows, pad), -1.0, scores.dtype)], axis=1)
        g = jnp.concatenate(
            [g, jnp.full((rows, pad), _BIG_G, g.dtype)], axis=1)
        zf = jnp.zeros((rows, pad), x1.dtype)
        x1 = jnp.concatenate([x1, zf], axis=1)
        y1 = jnp.concatenate([y1, zf], axis=1)
        x2 = jnp.concatenate([x2, zf], axis=1)
        y2 = jnp.concatenate([y2, zf], axis=1)
        prow = 16 - rows
        scores = jnp.concatenate(
            [scores, jnp.full((prow, 128), -1.0, scores.dtype)], axis=0)
        g = jnp.concatenate([g, jnp.full((prow, 128), _BIG_G, g.dtype)], axis=0)
        zr = jnp.zeros((prow, 128), x1.dtype)
        x1 = jnp.concatenate([x1, zr], axis=0)
        y1 = jnp.concatenate([y1, zr], axis=0)
        x2 = jnp.concatenate([x2, zr], axis=0)
        y2 = jnp.concatenate([y2, zr], axis=0)
    elif scores.shape[0] % 8 != 0:
        prow = 8 - scores.shape[0] % 8
        scores = jnp.concatenate(
            [scores, jnp.full((prow, 128), -1.0, scores.dtype)], axis=0)
        g = jnp.concatenate([g, jnp.full((prow, 128), _BIG_G, g.dtype)], axis=0)
        zr = jnp.zeros((prow, 128), x1.dtype)
        x1 = jnp.concatenate([x1, zr], axis=0)
        y1 = jnp.concatenate([y1, zr], axis=0)
        x2 = jnp.concatenate([x2, zr], axis=0)
        y2 = jnp.concatenate([y2, zr], axis=0)

    return scores, x1, y1, x2, y2, g


def _fused_kernel(c0, c1, c2, c3, c4, b0, b1, b2, b3, b4,
                  a0, a1, a2, a3, a4, out_ref):
    cls = [c0[...], c1[...], c2[...], c3[...], c4[...]]
    bbs = [b0[...], b1[...], b2[...], b3[...], b4[...]]
    ans = [a0[...], a1[...], a2[...], a3[...], a4[...]]

    parts = [_level_buffers(l, cls[l], bbs[l], ans[l]) for l in range(5)]
    scores = jnp.concatenate([pt[0] for pt in parts], axis=0)
    x1a = jnp.concatenate([pt[1] for pt in parts], axis=0)
    y1a = jnp.concatenate([pt[2] for pt in parts], axis=0)
    x2a = jnp.concatenate([pt[3] for pt in parts], axis=0)
    y2a = jnp.concatenate([pt[4] for pt in parts], axis=0)
    ga = jnp.concatenate([pt[5] for pt in parts], axis=0)
    areas = (x2a - x1a) * (y2a - y1a)

    lane = jax.lax.broadcasted_iota(jnp.int32, (1, 128), 1)

    def nms_body(t, sc):
        m = jnp.max(sc, keepdims=True)
        at_max = sc == m
        gm = jnp.min(jnp.where(at_max, ga, _BIG_G), keepdims=True)
        sel = (at_max & (ga == gm)).astype(jnp.float32)
        bx1 = jnp.sum(sel * x1a, keepdims=True)
        by1 = jnp.sum(sel * y1a, keepdims=True)
        bx2 = jnp.sum(sel * x2a, keepdims=True)
        by2 = jnp.sum(sel * y2a, keepdims=True)
        barea = jnp.sum(sel * areas, keepdims=True)

        xx1 = jnp.maximum(bx1, x1a)
        yy1 = jnp.maximum(by1, y1a)
        xx2 = jnp.minimum(bx2, x2a)
        yy2 = jnp.minimum(by2, y2a)
        inter = jnp.maximum(xx2 - xx1, 0.0) * jnp.maximum(yy2 - yy1, 0.0)
        iou = inter / (barea + areas - inter + 1e-6)
        sc_new = jnp.where(iou >= _IOU_THR, -1.0, sc)

        valid = m > _SCORE_THR
        row = jnp.where(lane == 0, bx1,
              jnp.where(lane == 1, by1,
              jnp.where(lane == 2, bx2,
              jnp.where(lane == 3, by2,
              jnp.where(lane == 4, m, 0.0)))))
        row = jnp.where(valid, row, 0.0)
        out_ref[pl.ds(t, 1), :] = row
        return sc_new

    jax.lax.fori_loop(0, _MAX_PER_IMG, nms_body, scores)


def _run(cls_list, bb_list, an_list):
    ins = []
    for l, (h, w) in enumerate(_FEAT_SIZES):
        hw = h * w
        if hw >= 128:
            ins.append(cls_list[l].reshape(_A, _NUM_CLASSES, hw // 128, 128))
        else:
            ins.append(cls_list[l].reshape(_A, _NUM_CLASSES, hw))
    for l, (h, w) in enumerate(_FEAT_SIZES):
        hw = h * w
        if hw >= 128:
            ins.append(bb_list[l].reshape(_A, 4, hw // 128, 128))
        else:
            ins.append(bb_list[l].reshape(_A, 4, hw))
    for l, (h, w) in enumerate(_FEAT_SIZES):
        hw = h * w
        an = an_list[l].reshape(hw, _A, 4).transpose(1, 2, 0)
        if hw >= 128:
            ins.append(an.reshape(_A, 4, hw // 128, 128))
        else:
            ins.append(an)
    out = pl.pallas_call(
        _fused_kernel,
        out_shape=jax.ShapeDtypeStruct((_MAX_PER_IMG, 128), jnp.float32),
        compiler_params=pltpu.CompilerParams(
            vmem_limit_bytes=100 * 1024 * 1024),
    )(*ins)
    return out[:, :5][None]


@jax.jit
def kernel(cls_score_0, cls_score_1, cls_score_2, cls_score_3, cls_score_4,
           bbox_pred_0, bbox_pred_1, bbox_pred_2, bbox_pred_3, bbox_pred_4,
           anchors_0, anchors_1, anchors_2, anchors_3, anchors_4):
    cls_list = [cls_score_0, cls_score_1, cls_score_2, cls_score_3, cls_score_4]
    bb_list = [bbox_pred_0, bbox_pred_1, bbox_pred_2, bbox_pred_3, bbox_pred_4]
    an_list = [anchors_0, anchors_1, anchors_2, anchors_3, anchors_4]
    return _run(cls_list, bb_list, an_list)


# speedup vs baseline: 1.1946x; 1.1946x over previous
import numpy as np
import jax
import jax.numpy as jnp
from jax.experimental import pallas as pl
from jax.experimental.pallas import tpu as pltpu

_FEAT_SIZES = [(64, 64), (32, 32), (16, 16), (8, 8), (4, 4)]
_NUM_CLASSES = 80
_A = 9
_IMG_H = 512.0
_IMG_W = 512.0
_NMS_PRE = 1000
_IOU_THR = 0.5
_SCORE_THR = 0.05
_MAX_PER_IMG = 100
_MAX_RATIO = float(np.abs(np.log(16.0 / 1000.0)))
_BIG_G = np.int32(2 ** 30)


def _level_buffers(l, cls, bb, an):
    h, w = _FEAT_SIZES[l]
    hw = h * w
    wide = hw >= 128
    if l == 0:
        mx = jnp.max(cls.reshape(_A, _NUM_CLASSES, h, w), axis=1)
        bb4 = bb.reshape(_A, 4, h, w)
        m = 128 // w
        S = _A * h
        Sm = S // m
        rows = Sm

        def _flat(x):
            x2 = x.reshape(S, w)
            return jnp.concatenate(
                [x2[j * Sm:(j + 1) * Sm] for j in range(m)], axis=1)

        comps = [_flat(z) for z in (
            mx, bb4[:, 0], bb4[:, 1], bb4[:, 2], bb4[:, 3],
            an[:, 0], an[:, 1], an[:, 2], an[:, 3])]
        maxlog, dx, dy, dw, dh, ax1, ay1, ax2, ay2 = comps
        shape = (rows, 128)
        ri = jax.lax.broadcasted_iota(jnp.int32, shape, 0)
        li = jax.lax.broadcasted_iota(jnp.int32, shape, 1)
        in_row = (li // w) * Sm + ri
        in_col = li % w
        a_idx = in_row // h
        y_idx = in_row % h
        p = (y_idx * w + in_col) * _A + a_idx
    else:
        mx = jnp.max(cls.reshape(_A, _NUM_CLASSES, hw), axis=1)
        bb4 = bb.reshape(_A, 4, hw)
        an4 = an.reshape(_A, 4, hw)
        if wide:
            m = hw // 128
            rows = _A * m

            def _flat(x):
                return jnp.concatenate(
                    [x[:, j * 128:(j + 1) * 128] for j in range(m)], axis=0)

            comps = [_flat(z) for z in (
                mx, bb4[:, 0], bb4[:, 1], bb4[:, 2], bb4[:, 3],
                an4[:, 0], an4[:, 1], an4[:, 2], an4[:, 3])]
            maxlog, dx, dy, dw, dh, ax1, ay1, ax2, ay2 = comps
            shape = (rows, 128)
            ri = jax.lax.broadcasted_iota(jnp.int32, shape, 0)
            li = jax.lax.broadcasted_iota(jnp.int32, shape, 1)
            pos = (ri // _A) * 128 + li
            a_idx = ri % _A
            p = pos * _A + a_idx
        else:
            rows = _A
            maxlog = mx
            dx, dy, dw, dh = bb4[:, 0], bb4[:, 1], bb4[:, 2], bb4[:, 3]
            ax1, ay1, ax2, ay2 = an4[:, 0], an4[:, 1], an4[:, 2], an4[:, 3]
            shape = (rows, hw)
            ri = jax.lax.broadcasted_iota(jnp.int32, shape, 0)
            li = jax.lax.broadcasted_iota(jnp.int32, shape, 1)
            p = li * _A + ri

    scores = jax.nn.sigmoid(maxlog)
    g = p + np.int32(l << 16)

    axc = (ax1 + ax2) * 0.5
    ayc = (ay1 + ay2) * 0.5
    aw = ax2 - ax1
    ah = ay2 - ay1
    dwc = jnp.clip(dw, -_MAX_RATIO, _MAX_RATIO)
    dhc = jnp.clip(dh, -_MAX_RATIO, _MAX_RATIO)
    cx = axc + aw * dx
    cy = ayc + ah * dy
    bw = aw * jnp.exp(dwc)
    bh = ah * jnp.exp(dhc)
    x1 = jnp.clip(cx - bw * 0.5, 0.0, _IMG_W)
    y1 = jnp.clip(cy - bh * 0.5, 0.0, _IMG_H)
    x2 = jnp.clip(cx + bw * 0.5, 0.0, _IMG_W)
    y2 = jnp.clip(cy + bh * 0.5, 0.0, _IMG_H)

    n = hw * _A
    k = min(_NMS_PRE, n)
    if k < n:
        keys = jax.lax.bitcast_convert_type(scores, jnp.int32)

        def bs_body(_, lohi):
            lo, hi = lohi
            mid = (lo + hi + 1) // 2
            c = jnp.sum((keys >= mid).astype(jnp.int32), keepdims=True)
            ge = c >= k
            return (jnp.where(ge, mid, lo), jnp.where(ge, hi, mid - 1))

        lo0 = jnp.zeros((1, 1), jnp.int32)
        hi0 = jnp.full((1, 1), 0x40000000, jnp.int32)
        T, _ = jax.lax.fori_loop(0, 31, bs_body, (lo0, hi0))

        strictly = keys > T
        m = jnp.sum(strictly.astype(jnp.int32), keepdims=True)
        need = k - m
        tie = keys == T

        def tie_body(_, lohi):
            lo, hi = lohi
            mid = (lo + hi) // 2
            c = jnp.sum((tie & (p <= mid)).astype(jnp.int32), keepdims=True)
            ge = c >= need
            return (jnp.where(ge, lo, mid + 1), jnp.where(ge, mid, hi))

        pstar, _ = jax.lax.fori_loop(
            0, 17, tie_body,
            (jnp.zeros((1, 1), jnp.int32), jnp.full((1, 1), n - 1, jnp.int32)))
        keep = strictly | (tie & (p <= pstar) & (need > 0))
        scores = jnp.where(keep, scores, -1.0)

    if not wide:
        pad = 128 - hw
        scores = jnp.concatenate(
            [scores, jnp.full((rows, pad), -1.0, scores.dtype)], axis=1)
        g = jnp.concatenate(
            [g, jnp.full((rows, pad), _BIG_G, g.dtype)], axis=1)
        zf = jnp.zeros((rows, pad), x1.dtype)
        x1 = jnp.concatenate([x1, zf], axis=1)
        y1 = jnp.concatenate([y1, zf], axis=1)
        x2 = jnp.concatenate([x2, zf], axis=1)
        y2 = jnp.concatenate([y2, zf], axis=1)
        prow = 16 - rows
        scores = jnp.concatenate(
            [scores, jnp.full((prow, 128), -1.0, scores.dtype)], axis=0)
        g = jnp.concatenate([g, jnp.full((prow, 128), _BIG_G, g.dtype)], axis=0)
        zr = jnp.zeros((prow, 128), x1.dtype)
        x1 = jnp.concatenate([x1, zr], axis=0)
        y1 = jnp.concatenate([y1, zr], axis=0)
        x2 = jnp.concatenate([x2, zr], axis=0)
        y2 = jnp.concatenate([y2, zr], axis=0)
    elif scores.shape[0] % 8 != 0:
        prow = 8 - scores.shape[0] % 8
        scores = jnp.concatenate(
            [scores, jnp.full((prow, 128), -1.0, scores.dtype)], axis=0)
        g = jnp.concatenate([g, jnp.full((prow, 128), _BIG_G, g.dtype)], axis=0)
        zr = jnp.zeros((prow, 128), x1.dtype)
        x1 = jnp.concatenate([x1, zr], axis=0)
        y1 = jnp.concatenate([y1, zr], axis=0)
        x2 = jnp.concatenate([x2, zr], axis=0)
        y2 = jnp.concatenate([y2, zr], axis=0)

    return scores, x1, y1, x2, y2, g


def _fused_kernel(c0, c1, c2, c3, c4, b0, b1, b2, b3, b4,
                  a0, a1, a2, a3, a4, out_ref):
    cls = [c0[...], c1[...], c2[...], c3[...], c4[...]]
    bbs = [b0[...], b1[...], b2[...], b3[...], b4[...]]
    ans = [a0[...], a1[...], a2[...], a3[...], a4[...]]

    parts = [_level_buffers(l, cls[l], bbs[l], ans[l]) for l in range(5)]
    scores = jnp.concatenate([pt[0] for pt in parts], axis=0)
    x1a = jnp.concatenate([pt[1] for pt in parts], axis=0)
    y1a = jnp.concatenate([pt[2] for pt in parts], axis=0)
    x2a = jnp.concatenate([pt[3] for pt in parts], axis=0)
    y2a = jnp.concatenate([pt[4] for pt in parts], axis=0)
    ga = jnp.concatenate([pt[5] for pt in parts], axis=0)
    areas = (x2a - x1a) * (y2a - y1a)

    lane = jax.lax.broadcasted_iota(jnp.int32, (1, 128), 1)

    def nms_body(t, sc):
        m = jnp.max(sc, keepdims=True)
        at_max = sc == m
        gm = jnp.min(jnp.where(at_max, ga, _BIG_G), keepdims=True)
        sel = (at_max & (ga == gm)).astype(jnp.float32)
        bx1 = jnp.sum(sel * x1a, keepdims=True)
        by1 = jnp.sum(sel * y1a, keepdims=True)
        bx2 = jnp.sum(sel * x2a, keepdims=True)
        by2 = jnp.sum(sel * y2a, keepdims=True)
        barea = jnp.sum(sel * areas, keepdims=True)

        xx1 = jnp.maximum(bx1, x1a)
        yy1 = jnp.maximum(by1, y1a)
        xx2 = jnp.minimum(bx2, x2a)
        yy2 = jnp.minimum(by2, y2a)
        inter = jnp.maximum(xx2 - xx1, 0.0) * jnp.maximum(yy2 - yy1, 0.0)
        iou = inter / (barea + areas - inter + 1e-6)
        sc_new = jnp.where(iou >= _IOU_THR, -1.0, sc)

        valid = m > _SCORE_THR
        row = jnp.where(lane == 0, bx1,
              jnp.where(lane == 1, by1,
              jnp.where(lane == 2, bx2,
              jnp.where(lane == 3, by2,
              jnp.where(lane == 4, m, 0.0)))))
        row = jnp.where(valid, row, 0.0)
        out_ref[pl.ds(t, 1), :] = row
        return sc_new

    jax.lax.fori_loop(0, _MAX_PER_IMG, nms_body, scores)


def _run(cls_list, bb_list, an_list):
    ins = []
    for l, (h, w) in enumerate(_FEAT_SIZES):
        if l == 0:
            ins.append(cls_list[l].reshape(_A * _NUM_CLASSES, h, w))
        else:
            ins.append(cls_list[l].reshape(_A * _NUM_CLASSES, h * w))
    for l, (h, w) in enumerate(_FEAT_SIZES):
        if l == 0:
            ins.append(bb_list[l].reshape(_A * 4, h, w))
        else:
            ins.append(bb_list[l].reshape(_A * 4, h * w))
    for l, (h, w) in enumerate(_FEAT_SIZES):
        hw = h * w
        if l == 0:
            ins.append(an_list[l].reshape(h, w, _A, 4).transpose(2, 3, 0, 1))
        else:
            ins.append(an_list[l].reshape(hw, _A, 4).transpose(1, 2, 0)
                       .reshape(_A * 4, hw))
    out = pl.pallas_call(
        _fused_kernel,
        out_shape=jax.ShapeDtypeStruct((_MAX_PER_IMG, 128), jnp.float32),
        compiler_params=pltpu.CompilerParams(
            vmem_limit_bytes=100 * 1024 * 1024),
    )(*ins)
    return out[:, :5][None]


@jax.jit
def kernel(cls_score_0, cls_score_1, cls_score_2, cls_score_3, cls_score_4,
           bbox_pred_0, bbox_pred_1, bbox_pred_2, bbox_pred_3, bbox_pred_4,
           anchors_0, anchors_1, anchors_2, anchors_3, anchors_4):
    cls_list = [cls_score_0, cls_score_1, cls_score_2, cls_score_3, cls_score_4]
    bb_list = [bbox_pred_0, bbox_pred_1, bbox_pred_2, bbox_pred_3, bbox_pred_4]
    an_list = [anchors_0, anchors_1, anchors_2, anchors_3, anchors_4]
    return _run(cls_list, bb_list, an_list)
